# baseline (speedup 1.0000x reference)
import functools

import jax
import jax.numpy as jnp
from jax import lax
from jax.experimental import pallas as pl
from jax.experimental.pallas import tpu as pltpu
from jax.experimental.pallas import tpu_sc as plsc

N = 10000
E = 320000
D = 128
H = 64

NC = 2
NS = 16
NW = NC * NS
CHUNK = 64
NCH = -(-E // (NW * CHUNK))
EPT = NCH * CHUNK
EPAD = NW * EPT
NPAD = 10240
RPT = NPAD // NS
ZROWS = RPT // 4

W1 = H + 16


def _make_seg_sum(width):
  mesh = plsc.VectorSubcoreMesh(core_axis_name="c", subcore_axis_name="s")

  @functools.partial(
      pl.kernel,
      out_type=jax.ShapeDtypeStruct((NC, NPAD, width), jnp.float32),
      mesh=mesh,
      scratch_types=[
          pltpu.VMEM_SHARED((NPAD, width), jnp.float32),
          pltpu.VMEM((NCH, CHUNK), jnp.int32),
          pltpu.VMEM((NCH, CHUNK), jnp.int32),
          [pltpu.VMEM((CHUNK, width), jnp.float32) for _ in range(4)],
          pltpu.VMEM((ZROWS, width), jnp.float32),
          [pltpu.SemaphoreType.DMA for _ in range(4)],
          [pltpu.SemaphoreType.DMA for _ in range(4)],
      ],
      compiler_params=pltpu.CompilerParams(use_tc_tiling_on_sc=False),
  )
  def seg_sum(tab_hbm, ei_hbm, out_hbm, acc_sh, srcv, dstv,
              rows, zbuf, gsem, ssem):
    c = lax.axis_index("c")
    s = lax.axis_index("s")
    wid = c * NS + s

    pltpu.sync_copy(ei_hbm.at[0, wid], srcv)
    pltpu.sync_copy(ei_hbm.at[1, wid], dstv)

    def zrow(r, carry):
      for j in range(width // 16):
        zbuf[r, pl.ds(j * 16, 16)] = jnp.zeros((16,), jnp.float32)
      return carry

    lax.fori_loop(0, ZROWS, zrow, 0)
    for b in range(RPT // ZROWS):
      pltpu.sync_copy(zbuf, acc_sh.at[pl.ds(s * RPT + b * ZROWS, ZROWS)])
    plsc.subcore_barrier()

    def gather(ch, b):
      pltpu.async_copy(tab_hbm.at[srcv.at[ch]], rows[b], gsem[b])

    def gwait(b):
      pltpu.make_async_copy(tab_hbm.at[srcv.at[0]], rows[b], gsem[b]).wait()

    def scat(ch, b):
      pltpu.async_copy(rows[b], acc_sh.at[dstv.at[ch]], ssem[b], add=True)

    def swait(b):
      pltpu.make_async_copy(rows[b], acc_sh.at[dstv.at[0]], ssem[b]).wait()

    gather(0, 0)
    gather(1, 1)

    def body(i, carry):
      for b in range(4):
        ch = 4 * i + b

        @pl.when(ch < NCH)
        def _():
          nb = (b + 2) % 4

          @pl.when(ch + 2 < NCH)
          def _():
            @pl.when(ch >= 2)
            def _():
              swait(nb)
            gather(ch + 2, nb)

          gwait(b)
          scat(ch, b)

      return carry

    lax.fori_loop(0, -(-NCH // 4), body, 0)
    for ch in range(max(0, NCH - 4), NCH):
      swait(ch % 4)
    plsc.subcore_barrier()
    pltpu.sync_copy(acc_sh.at[pl.ds(s * RPT, RPT)],
                    out_hbm.at[c, pl.ds(s * RPT, RPT)])

  return seg_sum


_seg_sum_w = _make_seg_sum(W1)
_seg_sum_h = _make_seg_sum(H)


def _dotT(a, b):
  return lax.dot_general(a, b, (((1,), (1,)), ((), ())),
                         preferred_element_type=jnp.float32)


def _tc0_body(x_ref, wl_ref, wr_ref, tab_ref, q_ref):
  x = x_ref[...]
  p = _dotT(x, wl_ref[...])
  ones_col = (lax.broadcasted_iota(jnp.int32, (N, W1 - H), 1) == 0)
  tab_ref[...] = jnp.concatenate([p, ones_col.astype(jnp.float32)], axis=1)
  q_ref[...] = _dotT(x, wr_ref[...])


_tc0 = pl.pallas_call(
    _tc0_body,
    out_shape=[
        jax.ShapeDtypeStruct((N, W1), jnp.float32),
        jax.ShapeDtypeStruct((N, H), jnp.float32),
    ],
)


def _tc1_body(parts_ref, q_ref, b_ref, wl_ref, wr_ref,
              tab_ref, q2_ref, inv_ref):
  m = parts_ref[0, :N, :] + parts_ref[1, :N, :]
  agg = m[:, :H]
  deg = jnp.sum(m[:, H:], axis=1, keepdims=True)
  inv = 1.0 / jnp.maximum(deg, 1.0)
  h = jnp.maximum(agg * inv + b_ref[...] + q_ref[...], 0.0)
  tab_ref[...] = _dotT(h, wl_ref[...])
  q2_ref[...] = _dotT(h, wr_ref[...])
  inv_ref[...] = inv


_tc1 = pl.pallas_call(
    _tc1_body,
    out_shape=[
        jax.ShapeDtypeStruct((N, H), jnp.float32),
        jax.ShapeDtypeStruct((N, H), jnp.float32),
        jax.ShapeDtypeStruct((N, 1), jnp.float32),
    ],
)


def _tc2_body(parts_ref, q_ref, inv_ref, b_ref, wl_ref, wr_ref,
              tab_ref, q2_ref):
  m = parts_ref[0, :N, :] + parts_ref[1, :N, :]
  h = jnp.maximum(m * inv_ref[...] + b_ref[...] + q_ref[...], 0.0)
  tab_ref[...] = _dotT(h, wl_ref[...])
  q2_ref[...] = _dotT(h, wr_ref[...])


_tc2 = pl.pallas_call(
    _tc2_body,
    out_shape=[
        jax.ShapeDtypeStruct((N, H), jnp.float32),
        jax.ShapeDtypeStruct((N, H), jnp.float32),
    ],
)


def _tc3_body(parts_ref, q_ref, inv_ref, b_ref, wc_ref, bc_ref, out_ref):
  m = parts_ref[0, :N, :] + parts_ref[1, :N, :]
  h = jnp.maximum(m * inv_ref[...] + b_ref[...] + q_ref[...], 0.0)
  out_ref[...] = _dotT(h, wc_ref[...])[:, :1] + bc_ref[0]


_tc3 = pl.pallas_call(
    _tc3_body,
    in_specs=[
        pl.BlockSpec(memory_space=pltpu.VMEM),
        pl.BlockSpec(memory_space=pltpu.VMEM),
        pl.BlockSpec(memory_space=pltpu.VMEM),
        pl.BlockSpec(memory_space=pltpu.VMEM),
        pl.BlockSpec(memory_space=pltpu.VMEM),
        pl.BlockSpec(memory_space=pltpu.SMEM),
    ],
    out_shape=jax.ShapeDtypeStruct((N, 1), jnp.float32),
)


@jax.jit
def kernel(x, edge_index, W1a_l, b1a, W1a_r, W1b_l, b1b, W1b_r,
           W2_l, b2, W2_r, Wc, bc):
  pad = jnp.tile(jnp.array([[0], [NPAD - 1]], jnp.int32), (1, EPAD - E))
  ei_flat = jnp.concatenate([edge_index, pad], axis=1).reshape(2, NW, NCH, CHUNK)
  tab1, q1 = _tc0(x, W1a_l, W1a_r)
  parts1 = _seg_sum_w(tab1, ei_flat)
  tab2, q2, inv = _tc1(parts1, q1, b1a.reshape(1, H), W1b_l, W1b_r)
  parts2 = _seg_sum_h(tab2, ei_flat)
  tab3, q3 = _tc2(parts2, q2, inv, b1b.reshape(1, H), W2_l, W2_r)
  parts3 = _seg_sum_h(tab3, ei_flat)
  wc_pad = jnp.zeros((128, H), jnp.float32).at[0].set(Wc[0])
  out = _tc3(parts3, q3, inv, b2.reshape(1, H), wc_pad, bc)
  return out[:, 0]


# speedup vs baseline: 1.5218x; 1.5218x over previous
import functools

import jax
import jax.numpy as jnp
from jax import lax
from jax.experimental import pallas as pl
from jax.experimental.pallas import tpu as pltpu
from jax.experimental.pallas import tpu_sc as plsc

N = 10000
E = 320000
D = 128
H = 64

NC = 2
NS = 16
NW = NC * NS
CHUNK = 80
NCH = -(-E // (NW * CHUNK))
EPT = NCH * CHUNK
EPAD = NW * EPT
NPAD = 10240
RPT = NPAD // NS
ZROWS = RPT // 8

W1 = H + 16

DEPTH = 4
NBUF = 2 * DEPTH


def _make_seg_sum(width):
  mesh = plsc.VectorSubcoreMesh(core_axis_name="c", subcore_axis_name="s")

  @functools.partial(
      pl.kernel,
      out_type=jax.ShapeDtypeStruct((NC, NPAD, width), jnp.float32),
      mesh=mesh,
      scratch_types=[
          pltpu.VMEM_SHARED((NPAD, width), jnp.float32),
          pltpu.VMEM((NCH, CHUNK), jnp.int32),
          pltpu.VMEM((NCH, CHUNK), jnp.int32),
          [pltpu.VMEM((CHUNK, width), jnp.float32) for _ in range(NBUF)],
          pltpu.VMEM((ZROWS, width), jnp.float32),
          [pltpu.SemaphoreType.DMA for _ in range(NBUF)],
          [pltpu.SemaphoreType.DMA for _ in range(NBUF)],
      ],
      compiler_params=pltpu.CompilerParams(use_tc_tiling_on_sc=False),
  )
  def seg_sum(tab_hbm, ei_hbm, out_hbm, acc_sh, srcv, dstv,
              rows, zbuf, gsem, ssem):
    c = lax.axis_index("c")
    s = lax.axis_index("s")
    wid = c * NS + s

    pltpu.sync_copy(ei_hbm.at[0, wid], srcv)
    pltpu.sync_copy(ei_hbm.at[1, wid], dstv)

    def zrow(r, carry):
      for j in range(width // 16):
        zbuf[r, pl.ds(j * 16, 16)] = jnp.zeros((16,), jnp.float32)
      return carry

    lax.fori_loop(0, ZROWS, zrow, 0)
    for b in range(RPT // ZROWS):
      pltpu.sync_copy(zbuf, acc_sh.at[pl.ds(s * RPT + b * ZROWS, ZROWS)])
    plsc.subcore_barrier()

    def gather(ch, b):
      pltpu.async_copy(tab_hbm.at[srcv.at[ch]], rows[b], gsem[b])

    def gwait(b):
      pltpu.make_async_copy(tab_hbm.at[srcv.at[0]], rows[b], gsem[b]).wait()

    def scat(ch, b):
      pltpu.async_copy(rows[b], acc_sh.at[dstv.at[ch]], ssem[b], add=True)

    def swait(b):
      pltpu.make_async_copy(rows[b], acc_sh.at[dstv.at[0]], ssem[b]).wait()

    for ch0 in range(DEPTH):
      gather(ch0, ch0)

    def body(i, carry):
      for b in range(NBUF):
        ch = NBUF * i + b

        @pl.when(ch < NCH)
        def _():
          nb = (b + DEPTH) % NBUF

          @pl.when(ch + DEPTH < NCH)
          def _():
            @pl.when(ch >= DEPTH)
            def _():
              swait(nb)
            gather(ch + DEPTH, nb)

          gwait(b)
          scat(ch, b)

      return carry

    lax.fori_loop(0, -(-NCH // NBUF), body, 0)
    for ch in range(max(0, NCH - NBUF), NCH):
      swait(ch % NBUF)
    plsc.subcore_barrier()
    pltpu.sync_copy(acc_sh.at[pl.ds(s * RPT, RPT)],
                    out_hbm.at[c, pl.ds(s * RPT, RPT)])

  return seg_sum


_seg_sum_w = _make_seg_sum(W1)
_seg_sum_h = _make_seg_sum(H)


def _dotT(a, b):
  return lax.dot_general(a, b, (((1,), (1,)), ((), ())),
                         preferred_element_type=jnp.float32)


def _tc0_body(x_ref, wl_ref, wr_ref, tab_ref, q_ref):
  x = x_ref[...]
  p = _dotT(x, wl_ref[...])
  ones_col = (lax.broadcasted_iota(jnp.int32, (N, W1 - H), 1) == 0)
  tab_ref[...] = jnp.concatenate([p, ones_col.astype(jnp.float32)], axis=1)
  q_ref[...] = _dotT(x, wr_ref[...])


_tc0 = pl.pallas_call(
    _tc0_body,
    out_shape=[
        jax.ShapeDtypeStruct((N, W1), jnp.float32),
        jax.ShapeDtypeStruct((N, H), jnp.float32),
    ],
)


def _tc1_body(parts_ref, q_ref, b_ref, wl_ref, wr_ref,
              tab_ref, q2_ref, inv_ref):
  m = parts_ref[0, :N, :] + parts_ref[1, :N, :]
  agg = m[:, :H]
  deg = jnp.sum(m[:, H:], axis=1, keepdims=True)
  inv = 1.0 / jnp.maximum(deg, 1.0)
  h = jnp.maximum(agg * inv + b_ref[...] + q_ref[...], 0.0)
  tab_ref[...] = _dotT(h, wl_ref[...])
  q2_ref[...] = _dotT(h, wr_ref[...])
  inv_ref[...] = inv


_tc1 = pl.pallas_call(
    _tc1_body,
    out_shape=[
        jax.ShapeDtypeStruct((N, H), jnp.float32),
        jax.ShapeDtypeStruct((N, H), jnp.float32),
        jax.ShapeDtypeStruct((N, 1), jnp.float32),
    ],
)


def _tc2_body(parts_ref, q_ref, inv_ref, b_ref, wl_ref, wr_ref,
              tab_ref, q2_ref):
  m = parts_ref[0, :N, :] + parts_ref[1, :N, :]
  h = jnp.maximum(m * inv_ref[...] + b_ref[...] + q_ref[...], 0.0)
  tab_ref[...] = _dotT(h, wl_ref[...])
  q2_ref[...] = _dotT(h, wr_ref[...])


_tc2 = pl.pallas_call(
    _tc2_body,
    out_shape=[
        jax.ShapeDtypeStruct((N, H), jnp.float32),
        jax.ShapeDtypeStruct((N, H), jnp.float32),
    ],
)


def _tc3_body(parts_ref, q_ref, inv_ref, b_ref, wc_ref, bc_ref, out_ref):
  m = parts_ref[0, :N, :] + parts_ref[1, :N, :]
  h = jnp.maximum(m * inv_ref[...] + b_ref[...] + q_ref[...], 0.0)
  out_ref[...] = _dotT(h, wc_ref[...])[:, :1] + bc_ref[0]


_tc3 = pl.pallas_call(
    _tc3_body,
    in_specs=[
        pl.BlockSpec(memory_space=pltpu.VMEM),
        pl.BlockSpec(memory_space=pltpu.VMEM),
        pl.BlockSpec(memory_space=pltpu.VMEM),
        pl.BlockSpec(memory_space=pltpu.VMEM),
        pl.BlockSpec(memory_space=pltpu.VMEM),
        pl.BlockSpec(memory_space=pltpu.SMEM),
    ],
    out_shape=jax.ShapeDtypeStruct((N, 1), jnp.float32),
)


@jax.jit
def kernel(x, edge_index, W1a_l, b1a, W1a_r, W1b_l, b1b, W1b_r,
           W2_l, b2, W2_r, Wc, bc):
  pad = jnp.tile(jnp.array([[0], [NPAD - 1]], jnp.int32), (1, EPAD - E))
  ei_flat = jnp.concatenate([edge_index, pad], axis=1).reshape(2, NW, NCH, CHUNK)
  tab1, q1 = _tc0(x, W1a_l, W1a_r)
  parts1 = _seg_sum_w(tab1, ei_flat)
  tab2, q2, inv = _tc1(parts1, q1, b1a.reshape(1, H), W1b_l, W1b_r)
  parts2 = _seg_sum_h(tab2, ei_flat)
  tab3, q3 = _tc2(parts2, q2, inv, b1b.reshape(1, H), W2_l, W2_r)
  parts3 = _seg_sum_h(tab3, ei_flat)
  wc_pad = jnp.zeros((128, H), jnp.float32).at[0].set(Wc[0])
  out = _tc3(parts3, q3, inv, b2.reshape(1, H), wc_pad, bc)
  return out[:, 0]


# speedup vs baseline: 1.5642x; 1.0278x over previous
import functools

import jax
import jax.numpy as jnp
from jax import lax
from jax.experimental import pallas as pl
from jax.experimental.pallas import tpu as pltpu
from jax.experimental.pallas import tpu_sc as plsc

N = 10000
E = 320000
D = 128
H = 64

NC = 2
NS = 16
NW = NC * NS
CHUNK = 80
NCH = -(-E // (NW * CHUNK))
EPT = NCH * CHUNK
EPAD = NW * EPT
NPAD = 10240
RPT = NPAD // NS
ZROWS = RPT // 8

W1 = H + 16

DEPTH = 4
NBUF = 2 * DEPTH


def _make_seg_sum(width):
  mesh = plsc.VectorSubcoreMesh(core_axis_name="c", subcore_axis_name="s")

  @functools.partial(
      pl.kernel,
      out_type=jax.ShapeDtypeStruct((NC, NPAD, width), jnp.float32),
      mesh=mesh,
      scratch_types=[
          pltpu.VMEM_SHARED((NPAD, width), jnp.float32),
          pltpu.VMEM((NCH, CHUNK), jnp.int32),
          pltpu.VMEM((NCH, CHUNK), jnp.int32),
          [pltpu.VMEM((CHUNK, width), jnp.float32) for _ in range(NBUF)],
          pltpu.VMEM((ZROWS, width), jnp.float32),
          [pltpu.SemaphoreType.DMA for _ in range(NBUF)],
          [pltpu.SemaphoreType.DMA for _ in range(NBUF)],
          pltpu.SemaphoreType.DMA,
          pltpu.SemaphoreType.DMA,
      ],
      compiler_params=pltpu.CompilerParams(use_tc_tiling_on_sc=False),
  )
  def seg_sum(tab_hbm, ei_hbm, out_hbm, acc_sh, srcv, dstv,
              rows, zbuf, gsem, ssem, isem, zsem):
    c = lax.axis_index("c")
    s = lax.axis_index("s")
    wid = c * NS + s

    pltpu.async_copy(ei_hbm.at[0, wid], srcv, isem)
    pltpu.async_copy(ei_hbm.at[1, wid], dstv, isem)

    def zrow(r, carry):
      for j in range(width // 16):
        zbuf[r, pl.ds(j * 16, 16)] = jnp.zeros((16,), jnp.float32)
      return carry

    lax.fori_loop(0, ZROWS, zrow, 0)
    for b in range(RPT // ZROWS):
      pltpu.async_copy(zbuf, acc_sh.at[pl.ds(s * RPT + b * ZROWS, ZROWS)],
                       zsem)
    for b in range(RPT // ZROWS):
      pltpu.make_async_copy(zbuf, acc_sh.at[pl.ds(s * RPT, ZROWS)],
                            zsem).wait()
    pltpu.make_async_copy(ei_hbm.at[0, wid], srcv, isem).wait()
    pltpu.make_async_copy(ei_hbm.at[1, wid], dstv, isem).wait()
    plsc.subcore_barrier()

    def gather(ch, b):
      pltpu.async_copy(tab_hbm.at[srcv.at[ch]], rows[b], gsem[b])

    def gwait(b):
      pltpu.make_async_copy(tab_hbm.at[srcv.at[0]], rows[b], gsem[b]).wait()

    def scat(ch, b):
      pltpu.async_copy(rows[b], acc_sh.at[dstv.at[ch]], ssem[b], add=True)

    def swait(b):
      pltpu.make_async_copy(rows[b], acc_sh.at[dstv.at[0]], ssem[b]).wait()

    for ch0 in range(DEPTH):
      gather(ch0, ch0)

    def body(i, carry):
      for b in range(NBUF):
        ch = NBUF * i + b

        @pl.when(ch < NCH)
        def _():
          nb = (b + DEPTH) % NBUF

          @pl.when(ch + DEPTH < NCH)
          def _():
            @pl.when(ch >= DEPTH)
            def _():
              swait(nb)
            gather(ch + DEPTH, nb)

          gwait(b)
          scat(ch, b)

      return carry

    lax.fori_loop(0, -(-NCH // NBUF), body, 0)
    for ch in range(max(0, NCH - NBUF), NCH):
      swait(ch % NBUF)
    plsc.subcore_barrier()
    pltpu.sync_copy(acc_sh.at[pl.ds(s * RPT, RPT)],
                    out_hbm.at[c, pl.ds(s * RPT, RPT)])

  return seg_sum


_seg_sum_w = _make_seg_sum(W1)
_seg_sum_h = _make_seg_sum(H)


def _dotT(a, b):
  return lax.dot_general(a, b, (((1,), (1,)), ((), ())),
                         preferred_element_type=jnp.float32)


def _tc0_body(x_ref, wl_ref, wr_ref, tab_ref, q_ref):
  x = x_ref[...]
  p = _dotT(x, wl_ref[...])
  ones_col = (lax.broadcasted_iota(jnp.int32, (N, W1 - H), 1) == 0)
  tab_ref[...] = jnp.concatenate([p, ones_col.astype(jnp.float32)], axis=1)
  q_ref[...] = _dotT(x, wr_ref[...])


_tc0 = pl.pallas_call(
    _tc0_body,
    out_shape=[
        jax.ShapeDtypeStruct((N, W1), jnp.float32),
        jax.ShapeDtypeStruct((N, H), jnp.float32),
    ],
)


def _tc1_body(parts_ref, q_ref, b_ref, wl_ref, wr_ref,
              tab_ref, q2_ref, inv_ref):
  m = parts_ref[0, :N, :] + parts_ref[1, :N, :]
  agg = m[:, :H]
  deg = jnp.sum(m[:, H:], axis=1, keepdims=True)
  inv = 1.0 / jnp.maximum(deg, 1.0)
  h = jnp.maximum(agg * inv + b_ref[...] + q_ref[...], 0.0)
  tab_ref[...] = _dotT(h, wl_ref[...])
  q2_ref[...] = _dotT(h, wr_ref[...])
  inv_ref[...] = inv


_tc1 = pl.pallas_call(
    _tc1_body,
    out_shape=[
        jax.ShapeDtypeStruct((N, H), jnp.float32),
        jax.ShapeDtypeStruct((N, H), jnp.float32),
        jax.ShapeDtypeStruct((N, 1), jnp.float32),
    ],
)


def _tc2_body(parts_ref, q_ref, inv_ref, b_ref, wl_ref, wr_ref,
              tab_ref, q2_ref):
  m = parts_ref[0, :N, :] + parts_ref[1, :N, :]
  h = jnp.maximum(m * inv_ref[...] + b_ref[...] + q_ref[...], 0.0)
  tab_ref[...] = _dotT(h, wl_ref[...])
  q2_ref[...] = _dotT(h, wr_ref[...])


_tc2 = pl.pallas_call(
    _tc2_body,
    out_shape=[
        jax.ShapeDtypeStruct((N, H), jnp.float32),
        jax.ShapeDtypeStruct((N, H), jnp.float32),
    ],
)


def _tc3_body(parts_ref, q_ref, inv_ref, b_ref, wc_ref, bc_ref, out_ref):
  m = parts_ref[0, :N, :] + parts_ref[1, :N, :]
  h = jnp.maximum(m * inv_ref[...] + b_ref[...] + q_ref[...], 0.0)
  out_ref[...] = _dotT(h, wc_ref[...])[:, :1] + bc_ref[0]


_tc3 = pl.pallas_call(
    _tc3_body,
    in_specs=[
        pl.BlockSpec(memory_space=pltpu.VMEM),
        pl.BlockSpec(memory_space=pltpu.VMEM),
        pl.BlockSpec(memory_space=pltpu.VMEM),
        pl.BlockSpec(memory_space=pltpu.VMEM),
        pl.BlockSpec(memory_space=pltpu.VMEM),
        pl.BlockSpec(memory_space=pltpu.SMEM),
    ],
    out_shape=jax.ShapeDtypeStruct((N, 1), jnp.float32),
)


@jax.jit
def kernel(x, edge_index, W1a_l, b1a, W1a_r, W1b_l, b1b, W1b_r,
           W2_l, b2, W2_r, Wc, bc):
  pad = jnp.tile(jnp.array([[0], [NPAD - 1]], jnp.int32), (1, EPAD - E))
  ei_flat = jnp.concatenate([edge_index, pad], axis=1).reshape(2, NW, NCH, CHUNK)
  tab1, q1 = _tc0(x, W1a_l, W1a_r)
  parts1 = _seg_sum_w(tab1, ei_flat)
  tab2, q2, inv = _tc1(parts1, q1, b1a.reshape(1, H), W1b_l, W1b_r)
  parts2 = _seg_sum_h(tab2, ei_flat)
  tab3, q3 = _tc2(parts2, q2, inv, b1b.reshape(1, H), W2_l, W2_r)
  parts3 = _seg_sum_h(tab3, ei_flat)
  wc_pad = jnp.zeros((128, H), jnp.float32).at[0].set(Wc[0])
  out = _tc3(parts3, q3, inv, b2.reshape(1, H), wc_pad, bc)
  return out[:, 0]
